# Initial kernel scaffold; baseline (speedup 1.0000x reference)
import jax
import jax.numpy as jnp
from jax.experimental import pallas as pl


def kernel(z, codebook):
    raise NotImplementedError("write your pallas kernel here")


# speedup vs baseline: 1.2211x; 1.2211x over previous
import jax
import jax.numpy as jnp
from jax import lax
from jax.experimental import pallas as pl
from jax.experimental.pallas import tpu as pltpu
from jax.experimental.pallas import tpu_sc as plsc

COMMITMENT_COST = 0.25

ROW_BLOCK = 256

_SC_CORES = 2
_SC_SUBCORES = 16
_NW = _SC_CORES * _SC_SUBCORES


def _dist_argmin_body(x_ref, cb_ref, idx_ref, loss_ref, e2_ref):
    i = pl.program_id(0)

    @pl.when(i == 0)
    def _():
        cb0 = cb_ref[...]
        e2_ref[...] = jnp.sum(cb0 ** 2, axis=1)

    x = x_ref[...]
    cb = cb_ref[...]
    x2 = jnp.sum(x ** 2, axis=1, keepdims=True)
    mm = lax.dot_general(x, cb, (((1,), (1,)), ((), ())))
    dists = x2 - 2.0 * mm + e2_ref[...][None, :]
    k = dists.shape[1]
    m = jnp.min(dists, axis=1)
    iota = lax.broadcasted_iota(jnp.int32, dists.shape, 1)
    idx_ref[0, 0, :] = jnp.min(jnp.where(dists == m[:, None], iota, k), axis=1)
    part = jnp.sum(m)

    @pl.when(i == 0)
    def _():
        loss_ref[0, 0] = part

    @pl.when(i != 0)
    def _():
        loss_ref[0, 0] += part


def _dist_argmin(flat, codebook):
    n, d = flat.shape
    k = codebook.shape[0]
    idx3, loss_sum = pl.pallas_call(
        _dist_argmin_body,
        grid=(n // ROW_BLOCK,),
        in_specs=[
            pl.BlockSpec((ROW_BLOCK, d), lambda i: (i, 0)),
            pl.BlockSpec((k, d), lambda i: (0, 0)),
        ],
        out_specs=[
            pl.BlockSpec((1, 1, ROW_BLOCK), lambda i: (i, 0, 0)),
            pl.BlockSpec(block_shape=(1, 1), index_map=lambda i: (0, 0),
                         memory_space=pltpu.SMEM),
        ],
        out_shape=[
            jax.ShapeDtypeStruct((n // ROW_BLOCK, 1, ROW_BLOCK), jnp.int32),
            jax.ShapeDtypeStruct((1, 1), jnp.float32),
        ],
        scratch_shapes=[pltpu.VMEM((k,), jnp.float32)],
    )(flat, codebook)
    return idx3.reshape(n), loss_sum[0, 0]


def _gather_body(cb_hbm, idx_hbm, out_hbm, idx_v, rows_v, sem):
    wid = lax.axis_index("s") * _SC_CORES + lax.axis_index("c")
    bpw = idx_v.shape[0]
    base = wid * bpw
    pltpu.sync_copy(idx_hbm.at[pl.ds(base, bpw)], idx_v)
    pltpu.async_copy(cb_hbm.at[idx_v], rows_v, sem).wait()
    pltpu.sync_copy(rows_v, out_hbm.at[pl.ds(base, bpw)])


def _gather_rows(codebook, idx):
    n = idx.shape[0]
    d = codebook.shape[1]
    bpw = n // _NW
    mesh = plsc.VectorSubcoreMesh(core_axis_name="c", subcore_axis_name="s")
    return pl.kernel(
        _gather_body,
        out_type=jax.ShapeDtypeStruct((n, d), jnp.float32),
        mesh=mesh,
        scratch_types=[
            pltpu.VMEM((bpw,), jnp.int32),
            pltpu.VMEM((bpw, d), jnp.float32),
            pltpu.SemaphoreType.DMA,
        ],
    )(codebook, idx)


def kernel(z, codebook):
    B, D, H, W = z.shape
    z_perm = jnp.transpose(z, (0, 2, 3, 1))
    flat = z_perm.reshape(-1, D)
    idx, loss_sum = _dist_argmin(flat, codebook)
    q = _gather_rows(codebook, idx)
    cl = loss_sum / jnp.float32(flat.size)
    loss = cl + COMMITMENT_COST * cl
    q_perm = q.reshape(B, H, W, D)
    q_st = z_perm + (q_perm - z_perm)
    quantized = jnp.transpose(q_st, (0, 3, 1, 2))
    indices = idx.reshape(B, H, W)
    return quantized, loss, indices


# speedup vs baseline: 1.2861x; 1.0532x over previous
import jax
import jax.numpy as jnp
from jax import lax
from jax.experimental import pallas as pl
from jax.experimental.pallas import tpu as pltpu
from jax.experimental.pallas import tpu_sc as plsc

COMMITMENT_COST = 0.25

ROW_BLOCK = 256

_SC_CORES = 2
_SC_SUBCORES = 16
_NW = _SC_CORES * _SC_SUBCORES
_LANES = 16


def _dist_argmin_body(x_ref, cb_ref, idx_ref, e2_ref):
    i = pl.program_id(0)

    @pl.when(i == 0)
    def _():
        cb0 = cb_ref[...]
        e2_ref[...] = jnp.sum(cb0 ** 2, axis=1)

    x = x_ref[...]
    cb = cb_ref[...]
    x2 = jnp.sum(x ** 2, axis=1, keepdims=True)
    mm = lax.dot_general(x, cb, (((1,), (1,)), ((), ())))
    dists = x2 - 2.0 * mm + e2_ref[...][None, :]
    idx_ref[0, 0, :] = jnp.argmin(dists, axis=1).astype(jnp.int32)


def _dist_argmin(flat, codebook):
    n, d = flat.shape
    k = codebook.shape[0]
    idx3 = pl.pallas_call(
        _dist_argmin_body,
        grid=(n // ROW_BLOCK,),
        in_specs=[
            pl.BlockSpec((ROW_BLOCK, d), lambda i: (i, 0)),
            pl.BlockSpec((k, d), lambda i: (0, 0)),
        ],
        out_specs=pl.BlockSpec((1, 1, ROW_BLOCK), lambda i: (i, 0, 0)),
        out_shape=jax.ShapeDtypeStruct((n // ROW_BLOCK, 1, ROW_BLOCK),
                                       jnp.int32),
        scratch_shapes=[pltpu.VMEM((k,), jnp.float32)],
    )(flat, codebook)
    return idx3.reshape(n)


def _gather_st_body(cb_hbm, idx_hbm, z_hbm, out_hbm, part_hbm,
                    idx_v, rows_v, z_v, acc_v, sem):
    wid = lax.axis_index("s") * _SC_CORES + lax.axis_index("c")
    bpw = idx_v.shape[0]
    d = rows_v.shape[1]
    base = wid * bpw
    pltpu.sync_copy(idx_hbm.at[pl.ds(base, bpw)], idx_v)
    pltpu.async_copy(cb_hbm.at[idx_v], rows_v, sem).wait()
    half = bpw // 2
    acc = jnp.zeros((_LANES,), jnp.float32)
    for c in range(2):
        pltpu.sync_copy(z_hbm.at[pl.ds(base + c * half, half)], z_v)

        def body(r, a, c=c):
            row = c * half + r
            for v in range(d // _LANES):
                qv = rows_v[row, pl.ds(v * _LANES, _LANES)]
                zv = z_v[r, pl.ds(v * _LANES, _LANES)]
                dv = qv - zv
                rows_v[row, pl.ds(v * _LANES, _LANES)] = zv + dv
                a = a + dv * dv
            return a

        acc = lax.fori_loop(0, half, body, acc)
    acc_v[...] = acc
    pltpu.sync_copy(rows_v, out_hbm.at[pl.ds(base, bpw)])
    pltpu.sync_copy(acc_v, part_hbm.at[wid])


def _gather_st(codebook, idx, flat):
    n = idx.shape[0]
    d = codebook.shape[1]
    bpw = n // _NW
    mesh = plsc.VectorSubcoreMesh(core_axis_name="c", subcore_axis_name="s")
    return pl.kernel(
        _gather_st_body,
        out_type=(jax.ShapeDtypeStruct((n, d), jnp.float32),
                  jax.ShapeDtypeStruct((_NW, _LANES), jnp.float32)),
        mesh=mesh,
        scratch_types=[
            pltpu.VMEM((bpw,), jnp.int32),
            pltpu.VMEM((bpw, d), jnp.float32),
            pltpu.VMEM((bpw // 2, d), jnp.float32),
            pltpu.VMEM((_LANES,), jnp.float32),
            pltpu.SemaphoreType.DMA,
        ],
    )(codebook, idx, flat)


def kernel(z, codebook):
    B, D, H, W = z.shape
    z_perm = jnp.transpose(z, (0, 2, 3, 1))
    flat = z_perm.reshape(-1, D)
    idx = _dist_argmin(flat, codebook)
    q_st, partials = _gather_st(codebook, idx, flat)
    cl = jnp.sum(partials) / jnp.float32(flat.size)
    loss = cl + COMMITMENT_COST * cl
    quantized = jnp.transpose(q_st.reshape(B, H, W, D), (0, 3, 1, 2))
    indices = idx.reshape(B, H, W)
    return quantized, loss, indices
